# baseline (speedup 1.0000x reference)
import functools

import jax
import jax.numpy as jnp
from jax import lax
from jax.experimental import pallas as pl
from jax.experimental.pallas import tpu as pltpu
from jax.experimental.pallas import tpu_sc as plsc

_AVG_RATING = 3.5
_NUM_CORES = 2
_NUM_SUBCORES = 16
_NW = _NUM_CORES * _NUM_SUBCORES


def _sc_gather4(user, item, gmf_user_table, gmf_item_table, mlp_user_table,
                mlp_item_table):
    B = user.shape[0]
    D = gmf_user_table.shape[1]
    bpw = B // _NW
    f32 = jnp.float32
    mesh = plsc.VectorSubcoreMesh(core_axis_name="c", subcore_axis_name="s")

    @functools.partial(
        pl.kernel,
        mesh=mesh,
        compiler_params=pltpu.CompilerParams(use_tc_tiling_on_sc=False),
        out_type=tuple(jax.ShapeDtypeStruct((B, D), f32) for _ in range(4)),
        scratch_types=[
            pltpu.VMEM((bpw,), jnp.int32),
            pltpu.VMEM((bpw,), jnp.int32),
            pltpu.VMEM((bpw, D), f32),
            pltpu.VMEM((bpw, D), f32),
            pltpu.VMEM((bpw, D), f32),
            pltpu.VMEM((bpw, D), f32),
            pltpu.SemaphoreType.DMA,
            pltpu.SemaphoreType.DMA,
            pltpu.SemaphoreType.DMA,
            pltpu.SemaphoreType.DMA,
        ],
    )
    def gather_kernel(u_hbm, i_hbm, gut, git, mut, mit, o0, o1, o2, o3,
                      iu, ii, r0, r1, r2, r3, s0, s1, s2, s3):
        wid = lax.axis_index("s") * _NUM_CORES + lax.axis_index("c")
        base = wid * bpw
        pltpu.sync_copy(u_hbm.at[pl.ds(base, bpw)], iu)
        pltpu.sync_copy(i_hbm.at[pl.ds(base, bpw)], ii)
        c0 = pltpu.async_copy(gut.at[iu], r0, s0)
        c1 = pltpu.async_copy(git.at[ii], r1, s1)
        c2 = pltpu.async_copy(mut.at[iu], r2, s2)
        c3 = pltpu.async_copy(mit.at[ii], r3, s3)
        c0.wait()
        pltpu.sync_copy(r0, o0.at[pl.ds(base, bpw)])
        c1.wait()
        pltpu.sync_copy(r1, o1.at[pl.ds(base, bpw)])
        c2.wait()
        pltpu.sync_copy(r2, o2.at[pl.ds(base, bpw)])
        c3.wait()
        pltpu.sync_copy(r3, o3.at[pl.ds(base, bpw)])

    return gather_kernel(user, item, gmf_user_table, gmf_item_table,
                         mlp_user_table, mlp_item_table)


def _dense_body(gu, gi, mu, mi, w1, w2, wf, bf, lab,
                pred_o, obj_o, mse_o):
    dim = gu.shape[1]
    dn = (((1,), (0,)), ((), ()))
    prec = lax.Precision.HIGHEST
    h = lax.dot_general(mu[...], w1[0:dim, :], dn, precision=prec,
                        preferred_element_type=jnp.float32)
    h += lax.dot_general(mi[...], w1[dim:2 * dim, :], dn, precision=prec,
                         preferred_element_type=jnp.float32)
    h = jnp.maximum(h, 0.0)
    h = lax.dot_general(h, w2[...], dn, precision=prec,
                        preferred_element_type=jnp.float32)
    h = jnp.maximum(h, 0.0)
    g = gu[...] * gi[...]
    pred = lax.dot_general(g, wf[0:dim, :], dn, precision=prec,
                           preferred_element_type=jnp.float32)
    pred += lax.dot_general(h, wf[dim:2 * dim, :], dn, precision=prec,
                            preferred_element_type=jnp.float32)
    pred = pred + (bf[0, 0] + _AVG_RATING)
    diff = pred - lab[...]
    mse = diff * diff
    pred_o[...] = pred
    mse_o[...] = mse
    obj_o[...] = jnp.sum(mse).reshape(1, 1)


def _dense(gu, gi, mu, mi, W1, W2, Wf, bf, label):
    B = gu.shape[0]
    f32 = jnp.float32
    out_shape = [
        jax.ShapeDtypeStruct((B, 1), f32),
        jax.ShapeDtypeStruct((1, 1), f32),
        jax.ShapeDtypeStruct((B, 1), f32),
    ]
    return pl.pallas_call(_dense_body, out_shape=out_shape)(
        gu, gi, mu, mi, W1, W2, Wf, bf.reshape(1, 1), label.reshape(B, 1))


def kernel(user, item, label, gmf_user_table, gmf_item_table, mlp_user_table,
           mlp_item_table, W1, W2, Wf, bf):
    user = user.astype(jnp.int32)
    item = item.astype(jnp.int32)
    gu, gi, mu, mi = _sc_gather4(user, item, gmf_user_table, gmf_item_table,
                                 mlp_user_table, mlp_item_table)
    pred, obj, mse = _dense(gu, gi, mu, mi, W1, W2, Wf, bf, label)
    return pred.reshape(-1), obj.reshape(()), mse.reshape(-1)


# speedup vs baseline: 1.1629x; 1.1629x over previous
import functools

import jax
import jax.numpy as jnp
from jax import lax
from jax.experimental import pallas as pl
from jax.experimental.pallas import tpu as pltpu
from jax.experimental.pallas import tpu_sc as plsc

_AVG_RATING = 3.5
_NUM_CORES = 2
_NUM_SUBCORES = 16
_NW = _NUM_CORES * _NUM_SUBCORES


def _sc_gather2(user, item, user_tab, item_tab):
    B = user.shape[0]
    D = user_tab.shape[1]
    bpw = B // _NW
    f32 = jnp.float32
    mesh = plsc.VectorSubcoreMesh(core_axis_name="c", subcore_axis_name="s")

    @functools.partial(
        pl.kernel,
        mesh=mesh,
        out_type=(jax.ShapeDtypeStruct((B, D), f32),
                  jax.ShapeDtypeStruct((B, D), f32)),
        scratch_types=[
            pltpu.VMEM((bpw,), jnp.int32),
            pltpu.VMEM((bpw,), jnp.int32),
            pltpu.VMEM((bpw, D), f32),
            pltpu.VMEM((bpw, D), f32),
            pltpu.SemaphoreType.DMA,
            pltpu.SemaphoreType.DMA,
        ],
    )
    def gather_kernel(u_hbm, i_hbm, ut, it, o0, o1, iu, ii, r0, r1, s0, s1):
        wid = lax.axis_index("s") * _NUM_CORES + lax.axis_index("c")
        base = wid * bpw
        pltpu.sync_copy(u_hbm.at[pl.ds(base, bpw)], iu)
        pltpu.sync_copy(i_hbm.at[pl.ds(base, bpw)], ii)
        c0 = pltpu.async_copy(ut.at[iu], r0, s0)
        c1 = pltpu.async_copy(it.at[ii], r1, s1)
        c0.wait()
        pltpu.sync_copy(r0, o0.at[pl.ds(base, bpw)])
        c1.wait()
        pltpu.sync_copy(r1, o1.at[pl.ds(base, bpw)])

    return gather_kernel(user, item, user_tab, item_tab)


def _dense_body(ur, ir, w1, w2, wf, bf, lab, pred_o, obj_o, mse_o):
    dim = ur.shape[1] // 2
    dn = (((1,), (0,)), ((), ()))
    prec = lax.Precision.HIGHEST
    gu = ur[:, 0:dim]
    mu = ur[:, dim:2 * dim]
    gi = ir[:, 0:dim]
    mi = ir[:, dim:2 * dim]
    h = lax.dot_general(mu, w1[0:dim, :], dn, precision=prec,
                        preferred_element_type=jnp.float32)
    h += lax.dot_general(mi, w1[dim:2 * dim, :], dn, precision=prec,
                         preferred_element_type=jnp.float32)
    h = jnp.maximum(h, 0.0)
    h = lax.dot_general(h, w2[...], dn, precision=prec,
                        preferred_element_type=jnp.float32)
    h = jnp.maximum(h, 0.0)
    g = gu * gi
    pred = lax.dot_general(g, wf[0:dim, :], dn, precision=prec,
                           preferred_element_type=jnp.float32)
    pred += lax.dot_general(h, wf[dim:2 * dim, :], dn, precision=prec,
                            preferred_element_type=jnp.float32)
    pred = pred + (bf[0, 0] + _AVG_RATING)
    diff = pred - lab[...]
    mse = diff * diff
    pred_o[...] = pred
    mse_o[...] = mse
    obj_o[...] = jnp.sum(mse).reshape(1, 1)


def _dense(ur, ir, W1, W2, Wf, bf, label):
    B = ur.shape[0]
    f32 = jnp.float32
    out_shape = [
        jax.ShapeDtypeStruct((B, 1), f32),
        jax.ShapeDtypeStruct((1, 1), f32),
        jax.ShapeDtypeStruct((B, 1), f32),
    ]
    return pl.pallas_call(_dense_body, out_shape=out_shape)(
        ur, ir, W1, W2, Wf, bf.reshape(1, 1), label.reshape(B, 1))


def kernel(user, item, label, gmf_user_table, gmf_item_table, mlp_user_table,
           mlp_item_table, W1, W2, Wf, bf):
    user = user.astype(jnp.int32)
    item = item.astype(jnp.int32)
    user_tab = jnp.concatenate([gmf_user_table, mlp_user_table], axis=1)
    item_tab = jnp.concatenate([gmf_item_table, mlp_item_table], axis=1)
    ur, ir = _sc_gather2(user, item, user_tab, item_tab)
    pred, obj, mse = _dense(ur, ir, W1, W2, Wf, bf, label)
    return pred.reshape(-1), obj.reshape(()), mse.reshape(-1)


# speedup vs baseline: 1.3928x; 1.1977x over previous
import functools

import jax
import jax.numpy as jnp
from jax import lax
from jax.experimental import pallas as pl
from jax.experimental.pallas import tpu as pltpu
from jax.experimental.pallas import tpu_sc as plsc

_AVG_RATING = 3.5
_NUM_CORES = 2
_NUM_SUBCORES = 16
_NW = _NUM_CORES * _NUM_SUBCORES


def _sc_gather4(user, item, gmf_user_table, gmf_item_table, mlp_user_table,
                mlp_item_table):
    B = user.shape[0]
    D = gmf_user_table.shape[1]
    bpw = B // _NW
    f32 = jnp.float32
    mesh = plsc.VectorSubcoreMesh(core_axis_name="c", subcore_axis_name="s")

    @functools.partial(
        pl.kernel,
        mesh=mesh,
        out_type=tuple(jax.ShapeDtypeStruct((B, D), f32) for _ in range(4)),
        scratch_types=[
            pltpu.VMEM((bpw,), jnp.int32),
            pltpu.VMEM((bpw,), jnp.int32),
            pltpu.VMEM((bpw, D), f32),
            pltpu.VMEM((bpw, D), f32),
            pltpu.VMEM((bpw, D), f32),
            pltpu.VMEM((bpw, D), f32),
            pltpu.SemaphoreType.DMA,
            pltpu.SemaphoreType.DMA,
            pltpu.SemaphoreType.DMA,
            pltpu.SemaphoreType.DMA,
        ],
    )
    def gather_kernel(u_hbm, i_hbm, t0, t1, t2, t3, o0, o1, o2, o3,
                      iu, ii, r0, r1, r2, r3, s0, s1, s2, s3):
        wid = lax.axis_index("s") * _NUM_CORES + lax.axis_index("c")
        base = wid * bpw
        pltpu.sync_copy(u_hbm.at[pl.ds(base, bpw)], iu)
        pltpu.sync_copy(i_hbm.at[pl.ds(base, bpw)], ii)

        @pl.loop(0, bpw, step=16)
        def _(g):
            vu = iu[pl.ds(g, 16)]
            vi = ii[pl.ds(g, 16)]
            for k in range(16):
                ju = vu[k]
                ji = vi[k]
                j = g + k
                pltpu.async_copy(t0.at[pl.ds(ju, 1)], r0.at[pl.ds(j, 1)], s0)
                pltpu.async_copy(t1.at[pl.ds(ji, 1)], r1.at[pl.ds(j, 1)], s1)
                pltpu.async_copy(t2.at[pl.ds(ju, 1)], r2.at[pl.ds(j, 1)], s2)
                pltpu.async_copy(t3.at[pl.ds(ji, 1)], r3.at[pl.ds(j, 1)], s3)

        pltpu.make_async_copy(t0.at[pl.ds(0, bpw)], r0, s0).wait()
        pltpu.make_async_copy(t1.at[pl.ds(0, bpw)], r1, s1).wait()
        pltpu.make_async_copy(t2.at[pl.ds(0, bpw)], r2, s2).wait()
        pltpu.make_async_copy(t3.at[pl.ds(0, bpw)], r3, s3).wait()

        pltpu.sync_copy(r0, o0.at[pl.ds(base, bpw)])
        pltpu.sync_copy(r1, o1.at[pl.ds(base, bpw)])
        pltpu.sync_copy(r2, o2.at[pl.ds(base, bpw)])
        pltpu.sync_copy(r3, o3.at[pl.ds(base, bpw)])

    return gather_kernel(user, item, gmf_user_table, gmf_item_table,
                         mlp_user_table, mlp_item_table)


def _dense_body(gu, gi, mu, mi, w1, w2, wf, bf, lab, pred_o, obj_o, mse_o):
    dim = gu.shape[1]
    dn = (((1,), (0,)), ((), ()))
    prec = lax.Precision.HIGHEST
    h = lax.dot_general(mu[...], w1[0:dim, :], dn, precision=prec,
                        preferred_element_type=jnp.float32)
    h += lax.dot_general(mi[...], w1[dim:2 * dim, :], dn, precision=prec,
                         preferred_element_type=jnp.float32)
    h = jnp.maximum(h, 0.0)
    h = lax.dot_general(h, w2[...], dn, precision=prec,
                        preferred_element_type=jnp.float32)
    h = jnp.maximum(h, 0.0)
    g = gu[...] * gi[...]
    pred = lax.dot_general(g, wf[0:dim, :], dn, precision=prec,
                           preferred_element_type=jnp.float32)
    pred += lax.dot_general(h, wf[dim:2 * dim, :], dn, precision=prec,
                            preferred_element_type=jnp.float32)
    pred = pred + (bf[0, 0] + _AVG_RATING)
    diff = pred - lab[...]
    mse = diff * diff
    pred_o[...] = pred
    mse_o[...] = mse
    obj_o[...] = jnp.sum(mse).reshape(1, 1)


def _dense(gu, gi, mu, mi, W1, W2, Wf, bf, label):
    B = gu.shape[0]
    f32 = jnp.float32
    out_shape = [
        jax.ShapeDtypeStruct((B, 1), f32),
        jax.ShapeDtypeStruct((1, 1), f32),
        jax.ShapeDtypeStruct((B, 1), f32),
    ]
    return pl.pallas_call(_dense_body, out_shape=out_shape)(
        gu, gi, mu, mi, W1, W2, Wf, bf.reshape(1, 1), label.reshape(B, 1))


def kernel(user, item, label, gmf_user_table, gmf_item_table, mlp_user_table,
           mlp_item_table, W1, W2, Wf, bf):
    user = user.astype(jnp.int32)
    item = item.astype(jnp.int32)
    gu, gi, mu, mi = _sc_gather4(user, item, gmf_user_table, gmf_item_table,
                                 mlp_user_table, mlp_item_table)
    pred, obj, mse = _dense(gu, gi, mu, mi, W1, W2, Wf, bf, label)
    return pred.reshape(-1), obj.reshape(()), mse.reshape(-1)


# speedup vs baseline: 1.6769x; 1.2039x over previous
import functools

import jax
import jax.numpy as jnp
from jax import lax
from jax.experimental import pallas as pl
from jax.experimental.pallas import tpu as pltpu
from jax.experimental.pallas import tpu_sc as plsc

_AVG_RATING = 3.5
_NUM_CORES = 2
_NUM_SUBCORES = 16
_NW = _NUM_CORES * _NUM_SUBCORES


def _pack_body(gT, mT, out):
    out[:, 0:64] = gT[...].T
    out[:, 64:128] = mT[...].T


def _pack_pair(gT, mT):
    V = gT.shape[1]
    nb = 3200
    grid = (pl.cdiv(V, nb),)
    return pl.pallas_call(
        _pack_body,
        grid=grid,
        in_specs=[
            pl.BlockSpec((64, nb), lambda i: (0, i)),
            pl.BlockSpec((64, nb), lambda i: (0, i)),
        ],
        out_specs=pl.BlockSpec((nb, 128), lambda i: (i, 0)),
        out_shape=jax.ShapeDtypeStruct((V, 128), jnp.float32),
    )(gT, mT)


def _sc_gather2(user, item, user_tab, item_tab):
    B = user.shape[0]
    D = user_tab.shape[1]
    bpw = B // _NW
    f32 = jnp.float32
    mesh = plsc.VectorSubcoreMesh(core_axis_name="c", subcore_axis_name="s")

    @functools.partial(
        pl.kernel,
        mesh=mesh,
        out_type=(jax.ShapeDtypeStruct((B, D), f32),
                  jax.ShapeDtypeStruct((B, D), f32)),
        scratch_types=[
            pltpu.VMEM((bpw,), jnp.int32),
            pltpu.VMEM((bpw,), jnp.int32),
            pltpu.VMEM((bpw, D), f32),
            pltpu.VMEM((bpw, D), f32),
            pltpu.SemaphoreType.DMA,
            pltpu.SemaphoreType.DMA,
        ],
    )
    def gather_kernel(u_hbm, i_hbm, ut, it, o0, o1, iu, ii, r0, r1, s0, s1):
        wid = lax.axis_index("s") * _NUM_CORES + lax.axis_index("c")
        base = wid * bpw
        pltpu.sync_copy(u_hbm.at[pl.ds(base, bpw)], iu)
        pltpu.sync_copy(i_hbm.at[pl.ds(base, bpw)], ii)
        c0 = pltpu.async_copy(ut.at[iu], r0, s0)
        c1 = pltpu.async_copy(it.at[ii], r1, s1)
        c0.wait()
        pltpu.sync_copy(r0, o0.at[pl.ds(base, bpw)])
        c1.wait()
        pltpu.sync_copy(r1, o1.at[pl.ds(base, bpw)])

    return gather_kernel(user, item, user_tab, item_tab)


def _dense_body(ur, ir, w1, w2, wf, bf, lab, pred_o, obj_o, mse_o):
    dim = ur.shape[1] // 2
    dn = (((1,), (0,)), ((), ()))
    prec = lax.Precision.HIGHEST
    gu = ur[:, 0:dim]
    mu = ur[:, dim:2 * dim]
    gi = ir[:, 0:dim]
    mi = ir[:, dim:2 * dim]
    h = lax.dot_general(mu, w1[0:dim, :], dn, precision=prec,
                        preferred_element_type=jnp.float32)
    h += lax.dot_general(mi, w1[dim:2 * dim, :], dn, precision=prec,
                         preferred_element_type=jnp.float32)
    h = jnp.maximum(h, 0.0)
    h = lax.dot_general(h, w2[...], dn, precision=prec,
                        preferred_element_type=jnp.float32)
    h = jnp.maximum(h, 0.0)
    g = gu * gi
    pred = lax.dot_general(g, wf[0:dim, :], dn, precision=prec,
                           preferred_element_type=jnp.float32)
    pred += lax.dot_general(h, wf[dim:2 * dim, :], dn, precision=prec,
                            preferred_element_type=jnp.float32)
    pred = pred + (bf[0, 0] + _AVG_RATING)
    diff = pred - lab[...]
    mse = diff * diff
    pred_o[...] = pred
    mse_o[...] = mse
    obj_o[...] = jnp.sum(mse).reshape(1, 1)


def _dense(ur, ir, W1, W2, Wf, bf, label):
    B = ur.shape[0]
    f32 = jnp.float32
    out_shape = [
        jax.ShapeDtypeStruct((B, 1), f32),
        jax.ShapeDtypeStruct((1, 1), f32),
        jax.ShapeDtypeStruct((B, 1), f32),
    ]
    return pl.pallas_call(_dense_body, out_shape=out_shape)(
        ur, ir, W1, W2, Wf, bf.reshape(1, 1), label.reshape(B, 1))


def kernel(user, item, label, gmf_user_table, gmf_item_table, mlp_user_table,
           mlp_item_table, W1, W2, Wf, bf):
    user = user.astype(jnp.int32)
    item = item.astype(jnp.int32)
    user_tab = _pack_pair(gmf_user_table.T, mlp_user_table.T)
    item_tab = _pack_pair(gmf_item_table.T, mlp_item_table.T)
    ur, ir = _sc_gather2(user, item, user_tab, item_tab)
    pred, obj, mse = _dense(ur, ir, W1, W2, Wf, bf, label)
    return pred.reshape(-1), obj.reshape(()), mse.reshape(-1)
